# Initial kernel scaffold; baseline (speedup 1.0000x reference)
import jax
import jax.numpy as jnp
from jax.experimental import pallas as pl


def kernel(feat0, feat1, feat2, mask_feat, adj0, adj1, mask_adj0, mask_adj1, nei0, nei1, W_fc0, b_fc0, W_fc1, b_fc1, W_fc2, b_fc2, W_agg0, W_agg1, W_g1, b_g1, W_g2, b_g2, W_att, b_att, a_att, W_proj, b_proj, W_m1, b_m1, W_m2, b_m2):
    raise NotImplementedError("write your pallas kernel here")


# speedup vs baseline: 5.2699x; 5.2699x over previous
import jax
import jax.numpy as jnp
from jax.experimental import pallas as pl

_N, _NA, _NS = 1024, 4096, 60
_H, _D = 256, 64
_TAU = 0.5


def _elu(x):
    return jnp.where(x > 0, x, jnp.exp(jnp.minimum(x, 0.0)) - 1.0)


def _dot(a, b):
    return jnp.dot(a, b, preferred_element_type=jnp.float32)


def _l2norm(x):
    n = jnp.sqrt(jnp.sum(x * x, axis=1, keepdims=True))
    return x / jnp.maximum(n, 1e-12)


def _lin_elu_kern(x_ref, w_ref, b_ref, o_ref):
    o_ref[...] = _elu(_dot(x_ref[...], w_ref[...]) + b_ref[...])


def _lin_elu(x, w, b, bm):
    m, k = x.shape
    n = w.shape[1]
    return pl.pallas_call(
        _lin_elu_kern,
        grid=(m // bm,),
        in_specs=[
            pl.BlockSpec((bm, k), lambda i: (i, 0)),
            pl.BlockSpec((k, n), lambda i: (0, 0)),
            pl.BlockSpec((1, n), lambda i: (0, 0)),
        ],
        out_specs=pl.BlockSpec((bm, n), lambda i: (i, 0)),
        out_shape=jax.ShapeDtypeStruct((m, n), jnp.float32),
    )(x, w, b)


def _agg_kern(nei_ref, h_ref, o_ref):
    nei = nei_ref[...]
    cnt = jnp.sum(nei, axis=1, keepdims=True)
    cnt = jnp.where(cnt > 0, cnt, 1.0)
    o_ref[...] = _dot(nei, h_ref[...]) / cnt


def _agg(nei, h, bm):
    m, k = nei.shape
    n = h.shape[1]
    return pl.pallas_call(
        _agg_kern,
        grid=(m // bm,),
        in_specs=[
            pl.BlockSpec((bm, k), lambda i: (i, 0)),
            pl.BlockSpec((k, n), lambda i: (0, 0)),
        ],
        out_specs=pl.BlockSpec((bm, n), lambda i: (i, 0)),
        out_shape=jax.ShapeDtypeStruct((m, n), jnp.float32),
    )(nei, h)


def _branch1_kern(f2_ref, w_ref, b_ref, nei_ref, o_ref):
    h = _elu(_dot(f2_ref[...], w_ref[...]) + b_ref[...])
    nei = nei_ref[...]
    cnt = jnp.sum(nei, axis=1, keepdims=True)
    cnt = jnp.where(cnt > 0, cnt, 1.0)
    o_ref[...] = _dot(nei, h) / cnt


def _views_kern(ht_ref, hm_ref, a0_ref, a1_ref, w0_ref, w1_ref,
                v0_ref, v1_ref, m0_ref, m1_ref):
    p0 = _dot(a0_ref[...], w0_ref[...])
    p1 = _dot(a1_ref[...], w1_ref[...])
    ht = ht_ref[...]
    hm = hm_ref[...]
    v0_ref[...] = _elu(ht + p0)
    v1_ref[...] = _elu(ht + p1)
    m0_ref[...] = _elu(hm + p0)
    m1_ref[...] = _elu(hm + p1)


def _gcn_coarse_kern(x_ref, a0_ref, a1_ref, w1_ref, b1_ref, w2_ref, b2_ref,
                     o_ref):
    adj = 0.5 * (a0_ref[...] + a1_ref[...])
    t = _dot(x_ref[...], w1_ref[...])
    h = jnp.maximum(_dot(adj, t) + b1_ref[...], 0.0)
    o_ref[...] = _dot(adj, _dot(h, w2_ref[...])) + b2_ref[...]


def _gcn_fine_kern(x_ref, adj_ref, w1_ref, b1_ref, w2_ref, b2_ref,
                   watt_ref, batt_ref, aatt_ref, hn_ref, s_ref):
    adj = adj_ref[...]
    t = _dot(x_ref[...], w1_ref[...])
    h = jnp.maximum(_dot(adj, t) + b1_ref[...], 0.0)
    z = _dot(adj, _dot(h, w2_ref[...])) + b2_ref[...]
    hn = _l2norm(z)
    s = _dot(jnp.tanh(_dot(hn, watt_ref[...]) + batt_ref[...]), aatt_ref[...])
    hn_ref[...] = hn
    s_ref[...] = (jnp.sum(s) / _N).reshape(1, 1)


def _proj_kern(zcr_ref, h0_ref, h1_ref, h2_ref, h3_ref, beta_ref,
               wp_ref, bp_ref, wm1_ref, bm1_ref,
               zf_ref, zc_ref, a_ref, bt_ref):
    b = beta_ref[...]
    zfine = (b[0, 0] * h0_ref[...] + b[0, 1] * h1_ref[...]
             + b[0, 2] * h2_ref[...] + b[0, 3] * h3_ref[...])

    def proj(z):
        return _l2norm(jnp.tanh(_dot(z, wp_ref[...]) + bp_ref[...]))

    zf = proj(zfine)
    zc = proj(zcr_ref[...])
    zf_ref[...] = zf
    zc_ref[...] = zc
    a_ref[...] = _dot(zf, wm1_ref[...])
    bt_ref[...] = (_dot(zc, wm1_ref[...]) + bm1_ref[...]).T


_BI = 128


def _pair_kern(zf_ref, zc_ref, a_ref, bt_ref, wm2_ref, bm2_ref,
               den_ref, dia_ref):
    i = pl.program_id(0)
    zf = zf_ref[...]
    zc = zc_ref[...]
    dots = _dot(zf, zc.T) * (1.0 / _TAU)
    a = a_ref[...]
    bt = bt_ref[...]
    acc = jnp.full((_BI, _N), bm2_ref[0, 0], jnp.float32)
    for k in range(16):
        acc = acc + wm2_ref[0, k] * jnp.tanh(a[:, k:k + 1] + bt[k:k + 1, :])
    weight = jax.nn.sigmoid(acc)
    den = jnp.sum(jnp.exp(dots) * weight, axis=1)
    cols = jax.lax.broadcasted_iota(jnp.int32, (_BI, _N), 1)
    rows = jax.lax.broadcasted_iota(jnp.int32, (_BI, _N), 0) + i * _BI
    dia = jnp.sum(jnp.where(cols == rows, dots, 0.0), axis=1)
    den_ref[...] = den.reshape(1, 1, _BI)
    dia_ref[...] = dia.reshape(1, 1, _BI)


def kernel(feat0, feat1, feat2, mask_feat, adj0, adj1, mask_adj0, mask_adj1,
           nei0, nei1, W_fc0, b_fc0, W_fc1, b_fc1, W_fc2, b_fc2,
           W_agg0, W_agg1, W_g1, b_g1, W_g2, b_g2, W_att, b_att, a_att,
           W_proj, b_proj, W_m1, b_m1, W_m2, b_m2):
    f0p = 1920 - feat0.shape[1]
    f1p = 384 - feat1.shape[1]
    x0 = jnp.concatenate([jnp.pad(feat0, ((0, 0), (0, f0p))),
                          jnp.pad(mask_feat, ((0, 0), (0, f0p)))], axis=0)
    wfc0 = jnp.pad(W_fc0, ((0, f0p), (0, 0)))
    feat1p = jnp.pad(feat1, ((0, 0), (0, f1p)))
    wfc1 = jnp.pad(W_fc1, ((0, f1p), (0, 0)))
    feat2p = jnp.pad(feat2, ((0, 64 - _NS), (0, 0)))
    nei1p = jnp.pad(nei1, ((0, 0), (0, 64 - _NS)))

    ht_hm = _lin_elu(x0, wfc0, b_fc0.reshape(1, -1), bm=256)
    h_tar, h_mask = ht_hm[:_N], ht_hm[_N:]
    h_nei0 = _lin_elu(feat1p, wfc1, b_fc1.reshape(1, -1), bm=512)

    agg0 = _agg(nei0, h_nei0, bm=256)
    agg1 = pl.pallas_call(
        _branch1_kern,
        out_shape=jax.ShapeDtypeStruct((_N, _H), jnp.float32),
    )(feat2p, W_fc2, b_fc2.reshape(1, -1), nei1p)

    h_view0, h_view1, h_mask0, h_mask1 = pl.pallas_call(
        _views_kern,
        grid=(4,),
        in_specs=[
            pl.BlockSpec((256, _H), lambda i: (i, 0)),
            pl.BlockSpec((256, _H), lambda i: (i, 0)),
            pl.BlockSpec((256, _H), lambda i: (i, 0)),
            pl.BlockSpec((256, _H), lambda i: (i, 0)),
            pl.BlockSpec((_H, _H), lambda i: (0, 0)),
            pl.BlockSpec((_H, _H), lambda i: (0, 0)),
        ],
        out_specs=[pl.BlockSpec((256, _H), lambda i: (i, 0))] * 4,
        out_shape=[jax.ShapeDtypeStruct((_N, _H), jnp.float32)] * 4,
    )(h_tar, h_mask, agg0, agg1, W_agg0, W_agg1)

    bg1 = b_g1.reshape(1, -1)
    bg2 = b_g2.reshape(1, -1)
    z_coarse = pl.pallas_call(
        _gcn_coarse_kern,
        out_shape=jax.ShapeDtypeStruct((_N, _D), jnp.float32),
    )(h_tar, adj0, adj1, W_g1, bg1, W_g2, bg2)

    def gcn_fine(x, adj):
        return pl.pallas_call(
            _gcn_fine_kern,
            out_shape=[jax.ShapeDtypeStruct((_N, _D), jnp.float32),
                       jax.ShapeDtypeStruct((1, 1), jnp.float32)],
        )(x, adj, W_g1, bg1, W_g2, bg2, W_att, b_att.reshape(1, -1),
          a_att.reshape(-1, 1))

    h0, s0 = gcn_fine(h_view0, adj0)
    h1, s1 = gcn_fine(h_mask0, mask_adj0)
    h2, s2 = gcn_fine(h_view1, adj1)
    h3, s3 = gcn_fine(h_mask1, mask_adj1)

    scores = jnp.concatenate([s0, s1, s2, s3], axis=1).reshape(4)
    beta = jax.nn.softmax(scores).reshape(1, 4)

    zf, zc, a_half, bt_half = pl.pallas_call(
        _proj_kern,
        out_shape=[jax.ShapeDtypeStruct((_N, _D), jnp.float32),
                   jax.ShapeDtypeStruct((_N, _D), jnp.float32),
                   jax.ShapeDtypeStruct((_N, 16), jnp.float32),
                   jax.ShapeDtypeStruct((16, _N), jnp.float32)],
    )(z_coarse, h0, h1, h2, h3, beta, W_proj, b_proj.reshape(1, -1),
      W_m1, b_m1.reshape(1, -1))

    den, dia = pl.pallas_call(
        _pair_kern,
        grid=(_N // _BI,),
        in_specs=[
            pl.BlockSpec((_BI, _D), lambda i: (i, 0)),
            pl.BlockSpec((_N, _D), lambda i: (0, 0)),
            pl.BlockSpec((_BI, 16), lambda i: (i, 0)),
            pl.BlockSpec((16, _N), lambda i: (0, 0)),
            pl.BlockSpec((1, 16), lambda i: (0, 0)),
            pl.BlockSpec((1, 1), lambda i: (0, 0)),
        ],
        out_specs=[pl.BlockSpec((1, 1, _BI), lambda i: (i, 0, 0))] * 2,
        out_shape=[jax.ShapeDtypeStruct((_N // _BI, 1, _BI), jnp.float32)] * 2,
    )(zf, zc, a_half, bt_half, W_m2.reshape(1, 16), b_m2.reshape(1, 1))

    den = den.reshape(_N)
    dia = dia.reshape(_N)
    return jnp.mean(jnp.log(den) - dia)
